# baseline (speedup 1.0000x reference)
import functools

import jax
import jax.numpy as jnp
from jax import lax
from jax.experimental import pallas as pl
from jax.experimental.pallas import tpu as pltpu
from jax.experimental.pallas import tpu_sc as plsc

N = 32768
H = 128
B = 16
DAPP = 32
NCLS = 2

NC = 2
NS = 16
NW = NC * NS
RW = N // NW
CH = 128
NCH = RW // CH
NBUF = 4
L = 16


NR = B + 1


def _seg_sum_body(h_hbm, seg_hbm, tgt_hbm, out_hbm,
                  seg_v, tgt_v, csg2_v, idx16_v, acc_v, bufe_v, bufo_v,
                  zero_v, acc_sh, ge_sem, go_sem, ss_sem):
    c = lax.axis_index("c")
    s = lax.axis_index("s")
    wid = s * NC + c
    base = wid * RW

    pltpu.sync_copy(seg_hbm.at[pl.ds(base, RW)], seg_v)
    pltpu.sync_copy(tgt_hbm.at[pl.ds(base, RW)], tgt_v)

    zv = jnp.zeros((L,), jnp.float32)
    lanes = lax.iota(jnp.int32, L)

    @pl.when(s == 0)
    def _zero():
        def zrow(i, carry):
            zero_v[i // (H // L), pl.ds((i % (H // L)) * L, L)] = zv
            return carry

        lax.fori_loop(0, NR * (H // L), zrow, 0)
        pltpu.sync_copy(zero_v, acc_sh)

    trash = jnp.zeros((L,), jnp.int32) + B

    def mkidx(i, carry):
        sv = seg_v[pl.ds(i * L, L)]
        tv = tgt_v[pl.ds(i * L, L)]
        csg2_v[i // (CH // L), pl.ds((i % (CH // L)) * L, L)] = jnp.where(
            tv == 1, sv, trash)
        return carry

    lax.fori_loop(0, RW // L, mkidx, 0)

    def zloc(i, carry):
        acc_v[i // (H // L), pl.ds((i % (H // L)) * L, L)] = zv
        return carry

    lax.fori_loop(0, B * (H // L), zloc, 0)
    idx16_v[0, pl.ds(0, L)] = lanes

    plsc.subcore_barrier()

    KE = NCH // 2
    for k in range(KE):
        pltpu.async_copy(h_hbm.at[pl.ds(base + 2 * k * CH, CH)],
                         bufe_v.at[k], ge_sem)
    for k in range(2):
        pltpu.async_copy(h_hbm.at[pl.ds(base + (2 * k + 1) * CH, CH)],
                         bufo_v.at[k], go_sem)

    for k in range(KE):
        ch = 2 * k
        pltpu.make_async_copy(h_hbm.at[pl.ds(base + ch * CH, CH)],
                              bufe_v.at[k], ge_sem).wait()
        pltpu.async_copy(bufe_v.at[k], acc_sh.at[csg2_v.at[ch]], ss_sem,
                         add=True)

    for k in range(KE):
        ch = 2 * k + 1
        so = k % 2
        pltpu.make_async_copy(h_hbm.at[pl.ds(base + ch * CH, CH)],
                              bufo_v.at[so], go_sem).wait()
        svf = seg_v[pl.ds(ch * CH, L)]
        svl = seg_v[pl.ds(ch * CH + CH - L, L)]
        uni = svf[0] == svl[L - 1]

        @pl.when(uni)
        def _uniform(ch=ch, so=so, svf=svf):
            def grp(g, acc):
                g0 = g * L
                tf = tgt_v[pl.ds(ch * CH + g0, L)].astype(jnp.float32)
                for r in range(L):
                    mf = tf[r]
                    acc = tuple(
                        acc[j] + bufo_v[so, g0 + r, pl.ds(j * L, L)] * mf
                        for j in range(H // L))
                return acc

            acc0 = tuple(zv for _ in range(H // L))
            accf = lax.fori_loop(0, CH // L, grp, acc0)
            row = svf[0]
            for j in range(H // L):
                plsc.addupdate(acc_v.at[row, pl.ds(j * L, L)], accf[j])

        @pl.when(jnp.logical_not(uni))
        def _mixed(ch=ch, so=so):
            pltpu.sync_copy(bufo_v.at[so], acc_sh.at[csg2_v.at[ch]],
                            add=True)

        if k + 2 < KE:
            pltpu.async_copy(
                h_hbm.at[pl.ds(base + (2 * (k + 2) + 1) * CH, CH)],
                bufo_v.at[so], go_sem)

    pltpu.sync_copy(acc_v, acc_sh.at[idx16_v.at[0]], add=True)
    for k in range(KE):
        pltpu.make_async_copy(bufe_v.at[k], acc_sh.at[csg2_v.at[2 * k]],
                              ss_sem).wait()

    plsc.subcore_barrier()

    @pl.when(s == 0)
    def _emit():
        pltpu.sync_copy(acc_sh, out_hbm.at[c])


@functools.lru_cache(maxsize=1)
def _seg_sum():
    return pl.kernel(
        _seg_sum_body,
        out_type=jax.ShapeDtypeStruct((NC, B + 1, H), jnp.float32),
        mesh=plsc.VectorSubcoreMesh(core_axis_name="c", subcore_axis_name="s"),
        scratch_types=[
            pltpu.VMEM((RW,), jnp.int32),
            pltpu.VMEM((RW,), jnp.int32),
            pltpu.VMEM((NCH, CH), jnp.int32),
            pltpu.VMEM((1, L), jnp.int32),
            pltpu.VMEM((B, H), jnp.float32),
            pltpu.VMEM((NCH // 2, CH, H), jnp.float32),
            pltpu.VMEM((2, CH, H), jnp.float32),
            pltpu.VMEM((NR, H), jnp.float32),
            pltpu.VMEM_SHARED((NR, H), jnp.float32),
            pltpu.SemaphoreType.DMA,
            pltpu.SemaphoreType.DMA,
            pltpu.SemaphoreType.DMA,
        ],
    )


def _stats_body(seg_ref, tgt_ref, dep_ref, feat_ref, w2_ref, w3_ref, b_ref,
                out_ref):
    seg = seg_ref[...]
    tgt = tgt_ref[...]
    dep = dep_ref[...]
    gid = lax.broadcasted_iota(jnp.int32, (B,) + seg.shape, 0)
    m = seg[None, :, :] == gid
    num_tot = jnp.sum(m.astype(jnp.float32), axis=(1, 2))
    num_tgt = jnp.sum(jnp.where(jnp.logical_and(m, tgt[None, :, :] == 1),
                                1.0, 0.0), axis=(1, 2))
    mx = jnp.max(jnp.where(m, dep[None, :, :], -jnp.inf), axis=(1, 2))
    out_ref[...] = (
        jnp.dot(feat_ref[...], w2_ref[...], preferred_element_type=jnp.float32)
        + mx[:, None] * w3_ref[0, :][None, :]
        + num_tgt[:, None] * w3_ref[1, :][None, :]
        + num_tot[:, None] * w3_ref[2, :][None, :]
        + b_ref[0, :][None, :]
    )


def _combine_body(parts_ref, rest_ref, w1_ref, out_ref):
    gh = parts_ref[0, :B, :] + parts_ref[1, :B, :]
    out_ref[...] = rest_ref[...] + jnp.dot(
        gh, w1_ref[...], preferred_element_type=jnp.float32)


def kernel(h, segment_ids, is_target, depth, feature, W, b):
    seg = segment_ids.astype(jnp.int32)
    tgt = is_target.astype(jnp.int32)
    parts = _seg_sum()(h, seg, tgt)
    rest = pl.pallas_call(
        _stats_body,
        out_shape=jax.ShapeDtypeStruct((B, NCLS), jnp.float32),
    )(seg.reshape(N // H, H), tgt.reshape(N // H, H),
      depth.reshape(N // H, H), feature,
      W[H:H + DAPP], W[H + DAPP:], b.reshape(1, NCLS))
    logits = pl.pallas_call(
        _combine_body,
        out_shape=jax.ShapeDtypeStruct((B, NCLS), jnp.float32),
    )(parts, rest, W[:H])
    return logits


# speedup vs baseline: 1.0542x; 1.0542x over previous
import functools

import jax
import jax.numpy as jnp
from jax import lax
from jax.experimental import pallas as pl
from jax.experimental.pallas import tpu as pltpu
from jax.experimental.pallas import tpu_sc as plsc

N = 32768
H = 128
B = 16
DAPP = 32
NCLS = 2

NC = 2
NS = 16
NW = NC * NS
RW = N // NW
CH = 128
NCH = RW // CH
NBUF = 4
L = 16


NR = B + 1


def _seg_sum_body(h_hbm, seg_hbm, tgt_hbm, out_hbm,
                  seg_v, tgt_v, csg2_v, idx16_v, acc_v, buf_v, zero_v,
                  acc_sh, gsem):
    c = lax.axis_index("c")
    s = lax.axis_index("s")
    wid = s * NC + c
    base = wid * RW

    pltpu.sync_copy(seg_hbm.at[pl.ds(base, RW)], seg_v)
    pltpu.sync_copy(tgt_hbm.at[pl.ds(base, RW)], tgt_v)

    zv = jnp.zeros((L,), jnp.float32)
    lanes = lax.iota(jnp.int32, L)

    @pl.when(s == 0)
    def _zero():
        def zrow(i, carry):
            zero_v[i // (H // L), pl.ds((i % (H // L)) * L, L)] = zv
            return carry

        lax.fori_loop(0, NR * (H // L), zrow, 0)
        pltpu.sync_copy(zero_v, acc_sh)

    trash = jnp.zeros((L,), jnp.int32) + B

    def mkidx(i, carry):
        sv = seg_v[pl.ds(i * L, L)]
        tv = tgt_v[pl.ds(i * L, L)]
        csg2_v[i // (CH // L), pl.ds((i % (CH // L)) * L, L)] = jnp.where(
            tv == 1, sv, trash)
        return carry

    lax.fori_loop(0, RW // L, mkidx, 0)

    def zloc(i, carry):
        acc_v[i // (H // L), pl.ds((i % (H // L)) * L, L)] = zv
        return carry

    lax.fori_loop(0, B * (H // L), zloc, 0)
    idx16_v[0, pl.ds(0, L)] = lanes

    plsc.subcore_barrier()

    pltpu.async_copy(h_hbm.at[pl.ds(base, CH)], buf_v.at[0], gsem)
    pltpu.async_copy(h_hbm.at[pl.ds(base + CH, CH)], buf_v.at[1], gsem)

    def chunk_body(i, carry):
        slot = lax.rem(i, 2)
        pltpu.make_async_copy(h_hbm.at[pl.ds(base + i * CH, CH)],
                              buf_v.at[slot], gsem).wait()
        svf = seg_v[pl.ds(i * CH, L)]
        svl = seg_v[pl.ds(i * CH + CH - L, L)]
        uni = svf[0] == svl[L - 1]

        @pl.when(uni)
        def _uniform():
            def grp(g, acc):
                g0 = g * L
                tf = tgt_v[pl.ds(i * CH + g0, L)].astype(jnp.float32)
                for r in range(L):
                    mf = tf[r]
                    acc = tuple(
                        acc[j] + buf_v[slot, g0 + r, pl.ds(j * L, L)] * mf
                        for j in range(H // L))
                return acc

            acc0 = tuple(zv for _ in range(H // L))
            accf = lax.fori_loop(0, CH // L, grp, acc0)
            row = svf[0]
            for j in range(H // L):
                plsc.addupdate(acc_v.at[row, pl.ds(j * L, L)], accf[j])

        @pl.when(jnp.logical_not(uni))
        def _mixed():
            pltpu.sync_copy(buf_v.at[slot], acc_sh.at[csg2_v.at[i]],
                            add=True)

        @pl.when(i + 2 < NCH)
        def _next():
            pltpu.async_copy(h_hbm.at[pl.ds(base + (i + 2) * CH, CH)],
                             buf_v.at[slot], gsem)

        return carry

    lax.fori_loop(0, NCH, chunk_body, 0)

    pltpu.sync_copy(acc_v, acc_sh.at[idx16_v.at[0]], add=True)

    plsc.subcore_barrier()

    @pl.when(s == 0)
    def _emit():
        pltpu.sync_copy(acc_sh, out_hbm.at[c])


@functools.lru_cache(maxsize=1)
def _seg_sum():
    return pl.kernel(
        _seg_sum_body,
        out_type=jax.ShapeDtypeStruct((NC, B + 1, H), jnp.float32),
        mesh=plsc.VectorSubcoreMesh(core_axis_name="c", subcore_axis_name="s"),
        scratch_types=[
            pltpu.VMEM((RW,), jnp.int32),
            pltpu.VMEM((RW,), jnp.int32),
            pltpu.VMEM((NCH, CH), jnp.int32),
            pltpu.VMEM((1, L), jnp.int32),
            pltpu.VMEM((B, H), jnp.float32),
            pltpu.VMEM((2, CH, H), jnp.float32),
            pltpu.VMEM((NR, H), jnp.float32),
            pltpu.VMEM_SHARED((NR, H), jnp.float32),
            pltpu.SemaphoreType.DMA,
        ],
    )


def _stats_body(seg_ref, tgt_ref, dep_ref, feat_ref, w2_ref, w3_ref, b_ref,
                out_ref):
    seg = seg_ref[...]
    tgt = tgt_ref[...]
    dep = dep_ref[...]
    gid = lax.broadcasted_iota(jnp.int32, (B,) + seg.shape, 0)
    m = seg[None, :, :] == gid
    num_tot = jnp.sum(m.astype(jnp.float32), axis=(1, 2))
    num_tgt = jnp.sum(jnp.where(jnp.logical_and(m, tgt[None, :, :] == 1),
                                1.0, 0.0), axis=(1, 2))
    mx = jnp.max(jnp.where(m, dep[None, :, :], -jnp.inf), axis=(1, 2))
    out_ref[...] = (
        jnp.dot(feat_ref[...], w2_ref[...], preferred_element_type=jnp.float32)
        + mx[:, None] * w3_ref[0, :][None, :]
        + num_tgt[:, None] * w3_ref[1, :][None, :]
        + num_tot[:, None] * w3_ref[2, :][None, :]
        + b_ref[0, :][None, :]
    )


def _combine_body(parts_ref, rest_ref, w1_ref, out_ref):
    gh = parts_ref[0, :B, :] + parts_ref[1, :B, :]
    out_ref[...] = rest_ref[...] + jnp.dot(
        gh, w1_ref[...], preferred_element_type=jnp.float32)


def kernel(h, segment_ids, is_target, depth, feature, W, b):
    seg = segment_ids.astype(jnp.int32)
    tgt = is_target.astype(jnp.int32)
    parts = _seg_sum()(h, seg, tgt)
    rest = pl.pallas_call(
        _stats_body,
        out_shape=jax.ShapeDtypeStruct((B, NCLS), jnp.float32),
    )(seg.reshape(N // H, H), tgt.reshape(N // H, H),
      depth.reshape(N // H, H), feature,
      W[H:H + DAPP], W[H + DAPP:], b.reshape(1, NCLS))
    logits = pl.pallas_call(
        _combine_body,
        out_shape=jax.ShapeDtypeStruct((B, NCLS), jnp.float32),
    )(parts, rest, W[:H])
    return logits


# speedup vs baseline: 1.0656x; 1.0108x over previous
import functools

import jax
import jax.numpy as jnp
from jax import lax
from jax.experimental import pallas as pl
from jax.experimental.pallas import tpu as pltpu
from jax.experimental.pallas import tpu_sc as plsc

N = 32768
H = 128
B = 16
DAPP = 32
NCLS = 2

NC = 2
NS = 16
NW = NC * NS
RW = N // NW
CH = 128
NCH = RW // CH
NBUF = 4
L = 16


NR = B + 1


def _seg_sum_body(h_hbm, seg_hbm, tgt_hbm, out_hbm,
                  seg_v, tgt_v, csg2_v, idx16_v, acc_v, buf_v, zero_v,
                  acc_sh, gsem):
    c = lax.axis_index("c")
    s = lax.axis_index("s")
    wid = s * NC + c
    base = wid * RW

    pltpu.sync_copy(seg_hbm.at[pl.ds(base, RW)], seg_v)
    pltpu.sync_copy(tgt_hbm.at[pl.ds(base, RW)], tgt_v)

    zv = jnp.zeros((L,), jnp.float32)
    lanes = lax.iota(jnp.int32, L)

    @pl.when(s == 0)
    def _zero():
        def zrow(i, carry):
            zero_v[i // (H // L), pl.ds((i % (H // L)) * L, L)] = zv
            return carry

        lax.fori_loop(0, NR * (H // L), zrow, 0)
        pltpu.sync_copy(zero_v, acc_sh)

    trash = jnp.zeros((L,), jnp.int32) + B

    def mkidx(i, carry):
        sv = seg_v[pl.ds(i * L, L)]
        tv = tgt_v[pl.ds(i * L, L)]
        csg2_v[i // (CH // L), pl.ds((i % (CH // L)) * L, L)] = jnp.where(
            tv == 1, sv, trash)
        return carry

    lax.fori_loop(0, RW // L, mkidx, 0)

    def zloc(i, carry):
        acc_v[i // (H // L), pl.ds((i % (H // L)) * L, L)] = zv
        return carry

    lax.fori_loop(0, B * (H // L), zloc, 0)
    idx16_v[0, pl.ds(0, L)] = lanes

    plsc.subcore_barrier()

    pltpu.async_copy(h_hbm.at[pl.ds(base, CH)], buf_v.at[0], gsem)
    pltpu.async_copy(h_hbm.at[pl.ds(base + CH, CH)], buf_v.at[1], gsem)

    def chunk_body(i, carry):
        slot = lax.rem(i, 2)
        pltpu.make_async_copy(h_hbm.at[pl.ds(base + i * CH, CH)],
                              buf_v.at[slot], gsem).wait()
        svf = seg_v[pl.ds(i * CH, L)]
        svl = seg_v[pl.ds(i * CH + CH - L, L)]
        uni = svf[0] == svl[L - 1]

        @pl.when(uni)
        def _uniform():
            def grp(g, acc):
                g0 = g * L
                tf = tgt_v[pl.ds(i * CH + g0, L)].astype(jnp.float32)
                ae, ao = acc
                for r in range(0, L, 2):
                    me = tf[r]
                    mo = tf[r + 1]
                    ae = tuple(
                        ae[j] + buf_v[slot, g0 + r, pl.ds(j * L, L)] * me
                        for j in range(H // L))
                    ao = tuple(
                        ao[j] + buf_v[slot, g0 + r + 1, pl.ds(j * L, L)] * mo
                        for j in range(H // L))
                return ae, ao

            acc0 = tuple(zv for _ in range(H // L))
            ae, ao = lax.fori_loop(0, CH // L, grp, (acc0, acc0))
            row = svf[0]
            for j in range(H // L):
                plsc.addupdate(acc_v.at[row, pl.ds(j * L, L)], ae[j] + ao[j])

        @pl.when(jnp.logical_not(uni))
        def _mixed():
            pltpu.sync_copy(buf_v.at[slot], acc_sh.at[csg2_v.at[i]],
                            add=True)

        @pl.when(i + 2 < NCH)
        def _next():
            pltpu.async_copy(h_hbm.at[pl.ds(base + (i + 2) * CH, CH)],
                             buf_v.at[slot], gsem)

        return carry

    lax.fori_loop(0, NCH, chunk_body, 0)

    pltpu.sync_copy(acc_v, acc_sh.at[idx16_v.at[0]], add=True)

    plsc.subcore_barrier()

    @pl.when(s == 0)
    def _emit():
        pltpu.sync_copy(acc_sh, out_hbm.at[c])


@functools.lru_cache(maxsize=1)
def _seg_sum():
    return pl.kernel(
        _seg_sum_body,
        out_type=jax.ShapeDtypeStruct((NC, B + 1, H), jnp.float32),
        mesh=plsc.VectorSubcoreMesh(core_axis_name="c", subcore_axis_name="s"),
        scratch_types=[
            pltpu.VMEM((RW,), jnp.int32),
            pltpu.VMEM((RW,), jnp.int32),
            pltpu.VMEM((NCH, CH), jnp.int32),
            pltpu.VMEM((1, L), jnp.int32),
            pltpu.VMEM((B, H), jnp.float32),
            pltpu.VMEM((2, CH, H), jnp.float32),
            pltpu.VMEM((NR, H), jnp.float32),
            pltpu.VMEM_SHARED((NR, H), jnp.float32),
            pltpu.SemaphoreType.DMA,
        ],
    )


def _stats_body(seg_ref, tgt_ref, dep_ref, feat_ref, w2_ref, w3_ref, b_ref,
                out_ref):
    seg = seg_ref[...]
    tgt = tgt_ref[...]
    dep = dep_ref[...]
    gid = lax.broadcasted_iota(jnp.int32, (B,) + seg.shape, 0)
    m = seg[None, :, :] == gid
    num_tot = jnp.sum(m.astype(jnp.float32), axis=(1, 2))
    num_tgt = jnp.sum(jnp.where(jnp.logical_and(m, tgt[None, :, :] == 1),
                                1.0, 0.0), axis=(1, 2))
    mx = jnp.max(jnp.where(m, dep[None, :, :], -jnp.inf), axis=(1, 2))
    out_ref[...] = (
        jnp.dot(feat_ref[...], w2_ref[...], preferred_element_type=jnp.float32)
        + mx[:, None] * w3_ref[0, :][None, :]
        + num_tgt[:, None] * w3_ref[1, :][None, :]
        + num_tot[:, None] * w3_ref[2, :][None, :]
        + b_ref[0, :][None, :]
    )


def _combine_body(parts_ref, rest_ref, w1_ref, out_ref):
    gh = parts_ref[0, :B, :] + parts_ref[1, :B, :]
    out_ref[...] = rest_ref[...] + jnp.dot(
        gh, w1_ref[...], preferred_element_type=jnp.float32)


def kernel(h, segment_ids, is_target, depth, feature, W, b):
    seg = segment_ids.astype(jnp.int32)
    tgt = is_target.astype(jnp.int32)
    parts = _seg_sum()(h, seg, tgt)
    rest = pl.pallas_call(
        _stats_body,
        out_shape=jax.ShapeDtypeStruct((B, NCLS), jnp.float32),
    )(seg.reshape(N // H, H), tgt.reshape(N // H, H),
      depth.reshape(N // H, H), feature,
      W[H:H + DAPP], W[H + DAPP:], b.reshape(1, NCLS))
    logits = pl.pallas_call(
        _combine_body,
        out_shape=jax.ShapeDtypeStruct((B, NCLS), jnp.float32),
    )(parts, rest, W[:H])
    return logits
